# baseline (speedup 1.0000x reference)
import functools
import jax
import jax.numpy as jnp
from jax import lax
from jax.experimental import pallas as pl
from jax.experimental.pallas import tpu as pltpu
from jax.experimental.pallas import tpu_sc as plsc

_NC = 2
_NS = 16
_NW = _NC * _NS
_D = 64
_SCALE = 8.0
_IDXROW = 128
_K = 4
_CROWS = _IDXROW * _K


def _make_kernel(B):
    bpw = B // _NW
    nrow = bpw // _IDXROW
    nchunk = bpw // _CROWS

    mesh = plsc.VectorSubcoreMesh(
        core_axis_name="c", subcore_axis_name="s",
        num_cores=_NC, num_subcores=_NS)

    @functools.partial(
        pl.kernel,
        out_type=jax.ShapeDtypeStruct((B, _D), jnp.float32),
        mesh=mesh,
        scratch_types=[
            pltpu.VMEM((nrow, _IDXROW), jnp.int32),
            pltpu.VMEM((_CROWS, _D), jnp.float32),
            pltpu.SemaphoreType.DMA,
        ],
        compiler_params=pltpu.CompilerParams(use_tc_tiling_on_sc=False),
    )
    def emb(idx_hbm, table_hbm, out_hbm, idx_v, rows, gsem):
        wid = lax.axis_index("s") * _NC + lax.axis_index("c")
        base = wid * bpw
        pltpu.sync_copy(idx_hbm.at[wid], idx_v)

        @pl.loop(0, nchunk)
        def chunk_loop(c):
            for k in range(_K):
                pltpu.async_copy(
                    table_hbm.at[idx_v.at[c * _K + k]],
                    rows.at[pl.ds(k * _IDXROW, _IDXROW)],
                    gsem)
            for k in range(_K):
                pltpu.make_async_copy(
                    table_hbm.at[idx_v.at[c * _K + k]],
                    rows.at[pl.ds(k * _IDXROW, _IDXROW)],
                    gsem).wait()

            @pl.loop(0, _CROWS)
            def scale_loop(r):
                for j in range(_D // 16):
                    s = pl.ds(j * 16, 16)
                    rows[r, s] = rows[r, s] * _SCALE

            pltpu.sync_copy(rows, out_hbm.at[pl.ds(base + c * _CROWS, _CROWS)])

    return emb


def kernel(batch_inputs, weight):
    bsz, seq = batch_inputs.shape
    B = bsz * seq
    idx = batch_inputs.astype(jnp.int32).reshape(_NW, B // (_NW * _IDXROW), _IDXROW)
    out = _make_kernel(B)(idx, weight)
    return out.reshape(bsz, seq, _D)


# speedup vs baseline: 1.1083x; 1.1083x over previous
import functools
import jax
import jax.numpy as jnp
from jax import lax
from jax.experimental import pallas as pl
from jax.experimental.pallas import tpu as pltpu
from jax.experimental.pallas import tpu_sc as plsc

_NC = 2
_NS = 16
_NW = _NC * _NS
_D = 64
_SCALE = 8.0
_IDXROW = 128
_NBUF = 4


def _make_kernel(B):
    bpw = B // _NW
    nchunk = bpw // _IDXROW

    mesh = plsc.VectorSubcoreMesh(
        core_axis_name="c", subcore_axis_name="s",
        num_cores=_NC, num_subcores=_NS)

    @functools.partial(
        pl.kernel,
        out_type=jax.ShapeDtypeStruct((B, _D), jnp.float32),
        mesh=mesh,
        scratch_types=[
            pltpu.VMEM((nchunk, _IDXROW), jnp.int32),
            [pltpu.VMEM((_IDXROW, _D), jnp.float32)] * _NBUF,
            [pltpu.SemaphoreType.DMA] * _NBUF,
            [pltpu.SemaphoreType.DMA] * _NBUF,
        ],
        compiler_params=pltpu.CompilerParams(use_tc_tiling_on_sc=False),
    )
    def emb(idx_hbm, table_hbm, out_hbm, idx_v, bufs, gsems, osems):
        wid = lax.axis_index("s") * _NC + lax.axis_index("c")
        base = wid * bpw
        pltpu.sync_copy(idx_hbm.at[wid], idx_v)

        def fire_gather(j, b):
            pltpu.async_copy(table_hbm.at[idx_v.at[j]], bufs[b], gsems[b])

        def wait_gather(j, b):
            pltpu.make_async_copy(
                table_hbm.at[idx_v.at[j]], bufs[b], gsems[b]).wait()

        def out_slice(j):
            return out_hbm.at[pl.ds(base + j * _IDXROW, _IDXROW)]

        for b in range(_NBUF - 1):
            fire_gather(b, b)

        @pl.loop(0, nchunk, step=_NBUF)
        def step(c):
            for db in range(_NBUF):
                j = c + db
                slot = db
                pb = (db + _NBUF - 1) % _NBUF
                wait_gather(j, slot)

                @pl.loop(0, _IDXROW)
                def scale_loop(r):
                    for u in range(_D // 16):
                        s = pl.ds(u * 16, 16)
                        bufs[slot][r, s] = bufs[slot][r, s] * _SCALE

                pltpu.async_copy(bufs[slot], out_slice(j), osems[slot])

                @pl.when(j + _NBUF - 1 < nchunk)
                def _():
                    @pl.when(j >= 1)
                    def _():
                        pltpu.make_async_copy(
                            bufs[pb], out_slice(j - 1), osems[pb]).wait()
                    fire_gather(j + _NBUF - 1, pb)

        for j in range(nchunk - _NBUF, nchunk):
            slot = j % _NBUF
            pltpu.make_async_copy(bufs[slot], out_slice(j), osems[slot]).wait()

    return emb


def kernel(batch_inputs, weight):
    bsz, seq = batch_inputs.shape
    B = bsz * seq
    idx = batch_inputs.astype(jnp.int32).reshape(_NW, B // (_NW * _IDXROW), _IDXROW)
    out = _make_kernel(B)(idx, weight)
    return out.reshape(bsz, seq, _D)
